# baseline (speedup 1.0000x reference)
import functools

import jax
import jax.numpy as jnp
from jax import lax
from jax.experimental import pallas as pl
from jax.experimental.pallas import tpu as pltpu
from jax.experimental.pallas import tpu_sc as plsc


def _sc_body(row_hbm, col_hbm, out_hbm, block, sem):
    wid = lax.axis_index("s") * 2 + lax.axis_index("c")
    fills = [
        pltpu.async_copy(col_hbm.at[pl.ds(0, 32), :],
                         block.at[:, pl.ds(0, 256)], sem)
    ]
    fills += [
        pltpu.async_copy(row_hbm.at[pl.ds(wid, 1), :],
                         block.at[pl.ds(j, 1), pl.ds(256, 256)], sem)
        for j in range(32)
    ]
    for cp in fills:
        cp.wait()
    p0 = wid * 32
    copies = [
        pltpu.async_copy(block, out_hbm.at[b, pl.ds(p0, 32), :], sem)
        for b in range(out_hbm.shape[0])
    ]
    for cp in copies:
        cp.wait()


def kernel(x, row_embed, col_embed):
    b = x.shape[0]
    run = pl.kernel(
        _sc_body,
        out_type=jax.ShapeDtypeStruct((b, 1024, 512), jnp.float32),
        mesh=plsc.VectorSubcoreMesh(core_axis_name="c", subcore_axis_name="s"),
        scratch_types=[
            pltpu.VMEM((32, 512), jnp.float32),
            pltpu.SemaphoreType.DMA,
        ],
    )
    out = run(row_embed, col_embed)
    return jnp.transpose(out.reshape(b, 32, 32, 512), (0, 3, 1, 2))


# speedup vs baseline: 1.1549x; 1.1549x over previous
import functools

import jax
import jax.numpy as jnp
from jax import lax
from jax.experimental import pallas as pl
from jax.experimental.pallas import tpu as pltpu
from jax.experimental.pallas import tpu_sc as plsc


def _sc_body(row_hbm, col_hbm, out_hbm, block, sem):
    wid = lax.axis_index("s") * 2 + lax.axis_index("c")
    fills = [
        pltpu.async_copy(col_hbm.at[pl.ds(0, 32), :],
                         block.at[:, pl.ds(0, 256)], sem),
        pltpu.async_copy(row_hbm.at[pl.ds(wid, 1), :],
                         block.at[pl.ds(0, 1), pl.ds(256, 256)], sem),
    ]
    for cp in fills:
        cp.wait()
    vals = [block[0, pl.ds(256 + 16 * k, 16)] for k in range(16)]
    for j in range(1, 32):
        for k in range(16):
            block[j, pl.ds(256 + 16 * k, 16)] = vals[k]
    p0 = wid * 32
    copies = [
        pltpu.async_copy(block, out_hbm.at[b, pl.ds(p0, 32), :], sem)
        for b in range(out_hbm.shape[0])
    ]
    for cp in copies:
        cp.wait()


def kernel(x, row_embed, col_embed):
    b = x.shape[0]
    run = pl.kernel(
        _sc_body,
        out_type=jax.ShapeDtypeStruct((b, 1024, 512), jnp.float32),
        mesh=plsc.VectorSubcoreMesh(core_axis_name="c", subcore_axis_name="s"),
        scratch_types=[
            pltpu.VMEM((32, 512), jnp.float32),
            pltpu.SemaphoreType.DMA,
        ],
    )
    out = run(row_embed, col_embed)
    return jnp.transpose(out.reshape(b, 32, 32, 512), (0, 3, 1, 2))


# speedup vs baseline: 5.0833x; 4.4013x over previous
import jax
import jax.numpy as jnp
from jax import lax
from jax.experimental import pallas as pl
from jax.experimental.pallas import tpu as pltpu


def _pos_body(row_ref, col_ref, out_ref, pos_vmem, sem):
    col32 = col_ref[0:32, :]
    row32 = row_ref[0:32, :]
    left = jnp.broadcast_to(col32[None, :, :], (32, 32, 256)).reshape(1024, 256)
    right = jnp.broadcast_to(row32[:, None, :], (32, 32, 256)).reshape(1024, 256)
    pos_vmem[:, 0:256] = left
    pos_vmem[:, 256:512] = right
    copies = [
        pltpu.make_async_copy(pos_vmem, out_ref.at[b], sem)
        for b in range(out_ref.shape[0])
    ]
    for cp in copies:
        cp.start()
    for cp in copies:
        cp.wait()


def kernel(x, row_embed, col_embed):
    b = x.shape[0]
    out = pl.pallas_call(
        _pos_body,
        in_specs=[
            pl.BlockSpec(memory_space=pltpu.MemorySpace.VMEM),
            pl.BlockSpec(memory_space=pltpu.MemorySpace.VMEM),
        ],
        out_specs=pl.BlockSpec(memory_space=pl.ANY),
        out_shape=jax.ShapeDtypeStruct((b, 1024, 512), jnp.float32),
        scratch_shapes=[
            pltpu.VMEM((1024, 512), jnp.float32),
            pltpu.SemaphoreType.DMA,
        ],
    )(row_embed, col_embed)
    return jnp.transpose(out.reshape(b, 32, 32, 512), (0, 3, 1, 2))
